# baseline (speedup 1.0000x reference)
import functools

import jax
import jax.numpy as jnp
import numpy as np
from jax import lax
from jax.experimental import pallas as pl
from jax.experimental.pallas import tpu as pltpu
from jax.experimental.pallas import tpu_sc as plsc

W, H = 128, 128
K = 7
HALF = K // 2
N = W * H
NUM_LONG = int(K * K * 0.2)
NUM_ITERS = 232
PROP_DIM = 64
NUM_MASKS = 30
MASK_THRESH = 0.5
L = 60
KT = K * K + NUM_LONG


def _local_indices():
    rr, cc = jnp.meshgrid(jnp.arange(H), jnp.arange(W), indexing='ij')
    offs = jnp.arange(-HALF, HALF + 1)
    dr, dc = jnp.meshgrid(offs, offs, indexing='ij')
    nr = jnp.clip(rr.reshape(-1, 1) + dr.reshape(1, -1), 0, H - 1)
    nc = jnp.clip(cc.reshape(-1, 1) + dc.reshape(1, -1), 0, W - 1)
    return (nr * W + nc).astype(jnp.int32).reshape(N, K * K)


def _graph():
    k1, k2, k3 = jax.random.split(jax.random.key(42), 3)
    local = _local_indices()
    long_rng = jax.random.randint(k1, (1, N, NUM_LONG), 0, N, dtype=jnp.int32)
    v = jnp.concatenate([local[None], long_rng], axis=-1)
    h0 = jax.nn.softmax(
        jax.random.normal(k2, (1, N, PROP_DIM), dtype=jnp.float32), axis=-1)
    agent_idx = jax.random.randint(k3, (1, NUM_MASKS), 0, N)
    return v[0], h0, agent_idx


E = N * KT
NW = 32
EPW = E // NW
CE = 1024
NCHUNK = EPW // CE

_SC_MESH = plsc.VectorSubcoreMesh(core_axis_name="c", subcore_axis_name="s")

_TVALS = tuple(float(x) for x in np.linspace(0.0, 1.0, L).astype(np.float32))


@functools.partial(
    pl.kernel,
    out_type=(
        jax.ShapeDtypeStruct((E,), jnp.float32),
        jax.ShapeDtypeStruct((E * L,), jnp.int32),
    ),
    mesh=_SC_MESH,
    compiler_params=pltpu.CompilerParams(needs_layout_passes=False),
    scratch_types=[
        pltpu.VMEM((N,), jnp.float32),
        pltpu.VMEM((CE,), jnp.int32),
        pltpu.VMEM((CE,), jnp.int32),
        pltpu.VMEM((CE,), jnp.float32),
        pltpu.VMEM((CE * L,), jnp.int32),
    ],
)
def _restrict_sc(u_hbm, v_hbm, b_hbm, red_hbm, inds_hbm,
                 btab, ubuf, vbuf, redbuf, indsbuf):
    wid = lax.axis_index("s") * 2 + lax.axis_index("c")
    base = wid * EPW
    pltpu.sync_copy(b_hbm, btab)

    def chunk_body(ci, _):
        cbase = base + ci * CE
        pltpu.sync_copy(u_hbm.at[pl.ds(cbase, CE)], ubuf)
        pltpu.sync_copy(v_hbm.at[pl.ds(cbase, CE)], vbuf)

        def grp(g, _):
            lane = lax.iota(jnp.int32, 16)
            lane60 = lane * L
            eloc = g * 16
            u = ubuf[pl.ds(eloc, 16)]
            vv = vbuf[pl.ds(eloc, 16)]
            fx = (u >> 7).astype(jnp.float32)
            fy = (u & (W - 1)).astype(jnp.float32)
            tx = (vv >> 7).astype(jnp.float32)
            ty = (vv & (W - 1)).astype(jnp.float32)
            dxf = tx - fx
            dyf = ty - fy
            red = jnp.zeros((16,), jnp.float32)
            sbase = eloc * L + lane60
            for l in range(L):
                tl = _TVALS[l]
                sx = (fx + dxf * tl).astype(jnp.int32)
                sy = (fy + dyf * tl).astype(jnp.int32)
                idx = (sx << 7) + sy
                red = red + plsc.load_gather(btab, [idx])
                plsc.store_scatter(indsbuf, [sbase + l], idx)
            redbuf[pl.ds(eloc, 16)] = red
            return 0

        lax.fori_loop(0, CE // 16, grp, 0)
        pltpu.sync_copy(redbuf, red_hbm.at[pl.ds(cbase, CE)])
        pltpu.sync_copy(indsbuf, inds_hbm.at[pl.ds(cbase * L, CE * L)])
        return 0

    lax.fori_loop(0, NCHUNK, chunk_body, 0)


EL = N * NUM_LONG
MPW = EL // NW
MCH = MPW // 128


@functools.partial(
    pl.kernel,
    out_type=(
        jax.ShapeDtypeStruct((N, PROP_DIM), jnp.float32),
        jax.ShapeDtypeStruct((N, PROP_DIM), jnp.float32),
    ),
    mesh=_SC_MESH,
    compiler_params=pltpu.CompilerParams(needs_layout_passes=False),
    scratch_types=[
        pltpu.VMEM_SHARED((N, PROP_DIM), jnp.float32),
        pltpu.VMEM((MCH, 128), jnp.int32),
        pltpu.VMEM((128, PROP_DIM), jnp.float32),
    ],
)
def _long_sc(msg_hbm, idx_hbm, zero_hbm, p0_hbm, p1_hbm, acc, idxbuf, msgbuf):
    cc = lax.axis_index("c")
    ss = lax.axis_index("s")
    wid = ss * 2 + cc
    base = wid * MPW
    pltpu.sync_copy(idx_hbm.at[wid], idxbuf)
    pltpu.sync_copy(zero_hbm.at[pl.ds(ss * (N // 16), N // 16)],
                    acc.at[pl.ds(ss * (N // 16), N // 16)])
    plsc.subcore_barrier()

    def chunk(r, _):
        pltpu.sync_copy(msg_hbm.at[pl.ds(base + r * 128, 128)], msgbuf)
        pltpu.sync_copy(msgbuf, acc.at[idxbuf.at[r]], add=True)
        return 0

    lax.fori_loop(0, MCH, chunk, 0)
    plsc.subcore_barrier()

    sl = pl.ds(ss * (N // 16), N // 16)

    @pl.when(cc == 0)
    def _():
        pltpu.sync_copy(acc.at[sl], p0_hbm.at[sl])

    @pl.when(cc == 1)
    def _():
        pltpu.sync_copy(acc.at[sl], p1_hbm.at[sl])


def _colshift_clip(y, dc):
    if dc == 0:
        return y
    pre = y.shape[:-1]
    if dc > 0:
        edge = jnp.sum(y[..., W - 1 - dc:], axis=-1, keepdims=True)
        return jnp.concatenate(
            [jnp.zeros(pre + (dc,), y.dtype), y[..., :W - 1 - dc], edge],
            axis=-1)
    d = -dc
    edge = jnp.sum(y[..., :d + 1], axis=-1, keepdims=True)
    return jnp.concatenate(
        [edge, y[..., d + 1:], jnp.zeros(pre + (d,), y.dtype)], axis=-1)


def _rowshift_clip(y, dr):
    if dr == 0:
        return y
    pre = y.shape[:-2]
    post = y.shape[-1:]
    if dr > 0:
        edge = jnp.sum(y[..., H - 1 - dr:, :], axis=-2, keepdims=True)
        return jnp.concatenate(
            [jnp.zeros(pre + (dr,) + post, y.dtype),
             y[..., :H - 1 - dr, :], edge], axis=-2)
    d = -dr
    edge = jnp.sum(y[..., :d + 1, :], axis=-2, keepdims=True)
    return jnp.concatenate(
        [edge, y[..., d + 1:, :], jnp.zeros(pre + (d,) + post, y.dtype)],
        axis=-2)


DB = 8


def _local_body(h_ref, wloc_ref, out_ref):
    h = h_ref[...]
    total = None
    k = 0
    for dr in range(-HALF, HALF + 1):
        acc = None
        for dc in range(-HALF, HALF + 1):
            y = wloc_ref[k][None] * h
            z = _colshift_clip(y, dc)
            acc = z if acc is None else acc + z
            k += 1
        r = _rowshift_clip(acc, dr)
        total = r if total is None else total + r
    out_ref[...] = total


def _softmax_body(x_ref, p_ref, hd_out_ref, hn_out_ref):
    x = x_ref[...] + p_ref[...]
    m = jnp.max(x, axis=0, keepdims=True)
    e = jnp.exp(x - m)
    s = jnp.sum(e, axis=0, keepdims=True)
    hd = e / s
    hd_out_ref[...] = hd
    hn_out_ref[...] = jnp.transpose(hd.reshape(PROP_DIM, N))


def _prop_step(hd, wloc, p_nodemajor):
    pd = jnp.transpose(p_nodemajor).reshape(PROP_DIM, H, W)
    pre = pl.pallas_call(
        _local_body,
        grid=(PROP_DIM // DB,),
        in_specs=[
            pl.BlockSpec((DB, H, W), lambda i: (i, 0, 0)),
            pl.BlockSpec((K * K, H, W), lambda i: (0, 0, 0)),
        ],
        out_specs=pl.BlockSpec((DB, H, W), lambda i: (i, 0, 0)),
        out_shape=jax.ShapeDtypeStruct((PROP_DIM, H, W), jnp.float32),
    )(hd, wloc)
    return pl.pallas_call(
        _softmax_body,
        out_shape=(
            jax.ShapeDtypeStruct((PROP_DIM, H, W), jnp.float32),
            jax.ShapeDtypeStruct((N, PROP_DIM), jnp.float32),
        ),
    )(pre, pd)


def _competition_body(flat_ref, agents_ref, aff_ref):
    flat = flat_ref[...]
    agents = agents_ref[...]
    scores = jax.lax.dot_general(
        flat, agents, (((1,), (1,)), ((), ())),
        preferred_element_type=jnp.float32)
    col = jax.lax.broadcasted_iota(jnp.int32, scores.shape, 1)
    valid = col < NUM_MASKS
    scores = jnp.where(valid, scores, -jnp.inf)
    m = jnp.max(scores, axis=-1, keepdims=True)
    e = jnp.where(valid, jnp.exp(scores - m), 0.0)
    masks = e / jnp.sum(e, axis=-1, keepdims=True)
    winner = jnp.max(masks, axis=0, keepdims=True)
    alive = jnp.where(winner > MASK_THRESH, 1.0, 0.0)
    aff_ref[...] = masks * alive


def _competition(flat, agents):
    agents_p = jnp.zeros((128, PROP_DIM), jnp.float32).at[:NUM_MASKS].set(agents)
    aff = pl.pallas_call(
        _competition_body,
        out_shape=jax.ShapeDtypeStruct((N, 128), jnp.float32),
    )(flat, agents_p)
    return aff[:, :NUM_MASKS]


def kernel(boundary):
    v, h0, agent_idx = _graph()

    u_flat = jnp.broadcast_to(
        jnp.arange(N, dtype=jnp.int32)[:, None], (N, KT)).reshape(-1)
    red, inds_flat = _restrict_sc(u_flat, v.reshape(-1), boundary.reshape(N))
    boundary_inds = inds_flat.reshape(1, N * KT, L)
    affinity = 10.0 - red.reshape(1, N, KT)

    adj = jax.nn.softmax(affinity, axis=-1)
    adj = adj / jnp.maximum(jnp.max(adj, axis=-1, keepdims=True), 1e-12)
    adj = adj[0]

    wloc = jnp.transpose(adj[:, :K * K]).reshape(K * K, H, W)
    wlong = adj[:, K * K:]
    vlong = v[:, K * K:].reshape(NW, MCH, 128)

    hd0 = jnp.transpose(h0[0]).reshape(PROP_DIM, H, W)
    zeros_nd = jnp.zeros((N, PROP_DIM), jnp.float32)

    def step(carry, _):
        hd, hn = carry
        msgs = (wlong[..., None] * hn[:, None, :]).reshape(-1, PROP_DIM)
        p0, p1 = _long_sc(msgs, vlong, zeros_nd)
        hd, hn = _prop_step(hd, wloc, p0 + p1)
        return (hd, hn), None

    (_, flat), _ = jax.lax.scan(step, (hd0, h0[0]), None, length=NUM_ITERS)

    prop_maps = flat[None]
    agents = jnp.take_along_axis(flat, agent_idx[0][:, None], axis=0)
    aff = _competition(flat, agents)
    aff_masks = aff.reshape(1, W, H, NUM_MASKS)
    return aff_masks, prop_maps, affinity, boundary_inds


# speedup vs baseline: 1.0057x; 1.0057x over previous
import functools

import jax
import jax.numpy as jnp
import numpy as np
from jax import lax
from jax.experimental import pallas as pl
from jax.experimental.pallas import tpu as pltpu
from jax.experimental.pallas import tpu_sc as plsc

W, H = 128, 128
K = 7
HALF = K // 2
N = W * H
NUM_LONG = int(K * K * 0.2)
NUM_ITERS = 232
PROP_DIM = 64
NUM_MASKS = 30
MASK_THRESH = 0.5
L = 60
KT = K * K + NUM_LONG


def _local_indices():
    rr, cc = jnp.meshgrid(jnp.arange(H), jnp.arange(W), indexing='ij')
    offs = jnp.arange(-HALF, HALF + 1)
    dr, dc = jnp.meshgrid(offs, offs, indexing='ij')
    nr = jnp.clip(rr.reshape(-1, 1) + dr.reshape(1, -1), 0, H - 1)
    nc = jnp.clip(cc.reshape(-1, 1) + dc.reshape(1, -1), 0, W - 1)
    return (nr * W + nc).astype(jnp.int32).reshape(N, K * K)


def _graph():
    k1, k2, k3 = jax.random.split(jax.random.key(42), 3)
    local = _local_indices()
    long_rng = jax.random.randint(k1, (1, N, NUM_LONG), 0, N, dtype=jnp.int32)
    v = jnp.concatenate([local[None], long_rng], axis=-1)
    h0 = jax.nn.softmax(
        jax.random.normal(k2, (1, N, PROP_DIM), dtype=jnp.float32), axis=-1)
    agent_idx = jax.random.randint(k3, (1, NUM_MASKS), 0, N)
    return v[0], h0, agent_idx


E = N * KT
NW = 32
EPW = E // NW
CE = 1024
NCHUNK = EPW // CE

_SC_MESH = plsc.VectorSubcoreMesh(core_axis_name="c", subcore_axis_name="s")

_TVALS = tuple(float(x) for x in np.linspace(0.0, 1.0, L).astype(np.float32))


@functools.partial(
    pl.kernel,
    out_type=(
        jax.ShapeDtypeStruct((E,), jnp.float32),
        jax.ShapeDtypeStruct((E * L,), jnp.int32),
    ),
    mesh=_SC_MESH,
    compiler_params=pltpu.CompilerParams(needs_layout_passes=False),
    scratch_types=[
        pltpu.VMEM((N,), jnp.float32),
        pltpu.VMEM((CE,), jnp.int32),
        pltpu.VMEM((CE,), jnp.int32),
        pltpu.VMEM((CE,), jnp.float32),
        pltpu.VMEM((CE * L,), jnp.int32),
    ],
)
def _restrict_sc(u_hbm, v_hbm, b_hbm, red_hbm, inds_hbm,
                 btab, ubuf, vbuf, redbuf, indsbuf):
    wid = lax.axis_index("s") * 2 + lax.axis_index("c")
    base = wid * EPW
    pltpu.sync_copy(b_hbm, btab)

    def chunk_body(ci, _):
        cbase = base + ci * CE
        pltpu.sync_copy(u_hbm.at[pl.ds(cbase, CE)], ubuf)
        pltpu.sync_copy(v_hbm.at[pl.ds(cbase, CE)], vbuf)

        def grp(g, _):
            lane = lax.iota(jnp.int32, 16)
            lane60 = lane * L
            eloc = g * 16
            u = ubuf[pl.ds(eloc, 16)]
            vv = vbuf[pl.ds(eloc, 16)]
            fx = (u >> 7).astype(jnp.float32)
            fy = (u & (W - 1)).astype(jnp.float32)
            tx = (vv >> 7).astype(jnp.float32)
            ty = (vv & (W - 1)).astype(jnp.float32)
            dxf = tx - fx
            dyf = ty - fy
            red = jnp.zeros((16,), jnp.float32)
            sbase = eloc * L + lane60
            for l in range(L):
                tl = _TVALS[l]
                sx = (fx + dxf * tl).astype(jnp.int32)
                sy = (fy + dyf * tl).astype(jnp.int32)
                idx = (sx << 7) + sy
                red = red + plsc.load_gather(btab, [idx])
                plsc.store_scatter(indsbuf, [sbase + l], idx)
            redbuf[pl.ds(eloc, 16)] = red
            return 0

        lax.fori_loop(0, CE // 16, grp, 0)
        pltpu.sync_copy(redbuf, red_hbm.at[pl.ds(cbase, CE)])
        pltpu.sync_copy(indsbuf, inds_hbm.at[pl.ds(cbase * L, CE * L)])
        return 0

    lax.fori_loop(0, NCHUNK, chunk_body, 0)


EL = N * NUM_LONG
MPW = EL // NW
MCH = MPW // 128


@functools.partial(
    pl.kernel,
    out_type=(
        jax.ShapeDtypeStruct((N, PROP_DIM), jnp.float32),
        jax.ShapeDtypeStruct((N, PROP_DIM), jnp.float32),
    ),
    mesh=_SC_MESH,
    compiler_params=pltpu.CompilerParams(needs_layout_passes=False),
    scratch_types=[
        pltpu.VMEM_SHARED((N, PROP_DIM), jnp.float32),
        pltpu.VMEM((MCH, 128), jnp.int32),
        pltpu.VMEM((128, PROP_DIM), jnp.float32),
    ],
)
def _long_sc(msg_hbm, idx_hbm, zero_hbm, p0_hbm, p1_hbm, acc, idxbuf, msgbuf):
    cc = lax.axis_index("c")
    ss = lax.axis_index("s")
    wid = ss * 2 + cc
    base = wid * MPW
    pltpu.sync_copy(idx_hbm.at[wid], idxbuf)
    pltpu.sync_copy(zero_hbm.at[pl.ds(ss * (N // 16), N // 16)],
                    acc.at[pl.ds(ss * (N // 16), N // 16)])
    plsc.subcore_barrier()

    def chunk(r, _):
        pltpu.sync_copy(msg_hbm.at[pl.ds(base + r * 128, 128)], msgbuf)
        pltpu.sync_copy(msgbuf, acc.at[idxbuf.at[r]], add=True)
        return 0

    lax.fori_loop(0, MCH, chunk, 0)
    plsc.subcore_barrier()

    sl = pl.ds(ss * (N // 16), N // 16)

    @pl.when(cc == 0)
    def _():
        pltpu.sync_copy(acc.at[sl], p0_hbm.at[sl])

    @pl.when(cc == 1)
    def _():
        pltpu.sync_copy(acc.at[sl], p1_hbm.at[sl])


def _colshift_clip(y, dc):
    if dc == 0:
        return y
    pre = y.shape[:-1]
    if dc > 0:
        edge = jnp.sum(y[..., W - 1 - dc:], axis=-1, keepdims=True)
        return jnp.concatenate(
            [jnp.zeros(pre + (dc,), y.dtype), y[..., :W - 1 - dc], edge],
            axis=-1)
    d = -dc
    edge = jnp.sum(y[..., :d + 1], axis=-1, keepdims=True)
    return jnp.concatenate(
        [edge, y[..., d + 1:], jnp.zeros(pre + (d,), y.dtype)], axis=-1)


def _rowshift_clip(y, dr):
    if dr == 0:
        return y
    pre = y.shape[:-2]
    post = y.shape[-1:]
    if dr > 0:
        edge = jnp.sum(y[..., H - 1 - dr:, :], axis=-2, keepdims=True)
        return jnp.concatenate(
            [jnp.zeros(pre + (dr,) + post, y.dtype),
             y[..., :H - 1 - dr, :], edge], axis=-2)
    d = -dr
    edge = jnp.sum(y[..., :d + 1, :], axis=-2, keepdims=True)
    return jnp.concatenate(
        [edge, y[..., d + 1:, :], jnp.zeros(pre + (d,) + post, y.dtype)],
        axis=-2)


DB = 8


def _local_body(h_ref, wloc_ref, out_ref):
    h = h_ref[...]
    first = True
    k = 0
    for dr in range(-HALF, HALF + 1):
        acc = None
        for dc in range(-HALF, HALF + 1):
            y = wloc_ref[k][None] * h
            z = _colshift_clip(y, dc)
            acc = z if acc is None else acc + z
            k += 1
        r = _rowshift_clip(acc, dr)
        if first:
            out_ref[...] = r
            first = False
        else:
            out_ref[...] += r


def _softmax_body(x_ref, p_ref, hd_out_ref, hn_out_ref):
    x = x_ref[...] + p_ref[...]
    m = jnp.max(x, axis=0, keepdims=True)
    e = jnp.exp(x - m)
    s = jnp.sum(e, axis=0, keepdims=True)
    hd = e / s
    hd_out_ref[...] = hd
    hn_out_ref[...] = jnp.transpose(hd.reshape(PROP_DIM, N))


def _prop_step(hd, wloc, p_nodemajor):
    pd = jnp.transpose(p_nodemajor).reshape(PROP_DIM, H, W)
    pre = pl.pallas_call(
        _local_body,
        grid=(PROP_DIM // DB,),
        in_specs=[
            pl.BlockSpec((DB, H, W), lambda i: (i, 0, 0)),
            pl.BlockSpec((K * K, H, W), lambda i: (0, 0, 0)),
        ],
        out_specs=pl.BlockSpec((DB, H, W), lambda i: (i, 0, 0)),
        out_shape=jax.ShapeDtypeStruct((PROP_DIM, H, W), jnp.float32),
    )(hd, wloc)
    return pl.pallas_call(
        _softmax_body,
        out_shape=(
            jax.ShapeDtypeStruct((PROP_DIM, H, W), jnp.float32),
            jax.ShapeDtypeStruct((N, PROP_DIM), jnp.float32),
        ),
    )(pre, pd)


def _competition_body(flat_ref, agents_ref, aff_ref):
    flat = flat_ref[...]
    agents = agents_ref[...]
    scores = jax.lax.dot_general(
        flat, agents, (((1,), (1,)), ((), ())),
        preferred_element_type=jnp.float32)
    col = jax.lax.broadcasted_iota(jnp.int32, scores.shape, 1)
    valid = col < NUM_MASKS
    scores = jnp.where(valid, scores, -jnp.inf)
    m = jnp.max(scores, axis=-1, keepdims=True)
    e = jnp.where(valid, jnp.exp(scores - m), 0.0)
    masks = e / jnp.sum(e, axis=-1, keepdims=True)
    winner = jnp.max(masks, axis=0, keepdims=True)
    alive = jnp.where(winner > MASK_THRESH, 1.0, 0.0)
    aff_ref[...] = masks * alive


def _competition(flat, agents):
    agents_p = jnp.zeros((128, PROP_DIM), jnp.float32).at[:NUM_MASKS].set(agents)
    aff = pl.pallas_call(
        _competition_body,
        out_shape=jax.ShapeDtypeStruct((N, 128), jnp.float32),
    )(flat, agents_p)
    return aff[:, :NUM_MASKS]


def kernel(boundary):
    v, h0, agent_idx = _graph()

    u_flat = jnp.broadcast_to(
        jnp.arange(N, dtype=jnp.int32)[:, None], (N, KT)).reshape(-1)
    red, inds_flat = _restrict_sc(u_flat, v.reshape(-1), boundary.reshape(N))
    boundary_inds = inds_flat.reshape(1, N * KT, L)
    affinity = 10.0 - red.reshape(1, N, KT)

    adj = jax.nn.softmax(affinity, axis=-1)
    adj = adj / jnp.maximum(jnp.max(adj, axis=-1, keepdims=True), 1e-12)
    adj = adj[0]

    wloc = jnp.transpose(adj[:, :K * K]).reshape(K * K, H, W)
    wlong = adj[:, K * K:]
    vlong = v[:, K * K:].reshape(NW, MCH, 128)

    hd0 = jnp.transpose(h0[0]).reshape(PROP_DIM, H, W)
    zeros_nd = jnp.zeros((N, PROP_DIM), jnp.float32)

    def step(carry, _):
        hd, hn = carry
        msgs = (wlong[..., None] * hn[:, None, :]).reshape(-1, PROP_DIM)
        p0, p1 = _long_sc(msgs, vlong, zeros_nd)
        hd, hn = _prop_step(hd, wloc, p0 + p1)
        return (hd, hn), None

    (_, flat), _ = jax.lax.scan(step, (hd0, h0[0]), None, length=NUM_ITERS)

    prop_maps = flat[None]
    agents = jnp.take_along_axis(flat, agent_idx[0][:, None], axis=0)
    aff = _competition(flat, agents)
    aff_masks = aff.reshape(1, W, H, NUM_MASKS)
    return aff_masks, prop_maps, affinity, boundary_inds


# speedup vs baseline: 1.0129x; 1.0072x over previous
import functools

import jax
import jax.numpy as jnp
import numpy as np
from jax import lax
from jax.experimental import pallas as pl
from jax.experimental.pallas import tpu as pltpu
from jax.experimental.pallas import tpu_sc as plsc

W, H = 128, 128
K = 7
HALF = K // 2
N = W * H
NUM_LONG = int(K * K * 0.2)
NUM_ITERS = 232
PROP_DIM = 64
NUM_MASKS = 30
MASK_THRESH = 0.5
L = 60
KT = K * K + NUM_LONG


def _local_indices():
    rr, cc = jnp.meshgrid(jnp.arange(H), jnp.arange(W), indexing='ij')
    offs = jnp.arange(-HALF, HALF + 1)
    dr, dc = jnp.meshgrid(offs, offs, indexing='ij')
    nr = jnp.clip(rr.reshape(-1, 1) + dr.reshape(1, -1), 0, H - 1)
    nc = jnp.clip(cc.reshape(-1, 1) + dc.reshape(1, -1), 0, W - 1)
    return (nr * W + nc).astype(jnp.int32).reshape(N, K * K)


def _graph():
    k1, k2, k3 = jax.random.split(jax.random.key(42), 3)
    local = _local_indices()
    long_rng = jax.random.randint(k1, (1, N, NUM_LONG), 0, N, dtype=jnp.int32)
    v = jnp.concatenate([local[None], long_rng], axis=-1)
    h0 = jax.nn.softmax(
        jax.random.normal(k2, (1, N, PROP_DIM), dtype=jnp.float32), axis=-1)
    agent_idx = jax.random.randint(k3, (1, NUM_MASKS), 0, N)
    return v[0], h0, agent_idx


E = N * KT
NW = 32
EPW = E // NW
CE = 1024
NCHUNK = EPW // CE

_SC_MESH = plsc.VectorSubcoreMesh(core_axis_name="c", subcore_axis_name="s")

_TVALS = tuple(float(x) for x in np.linspace(0.0, 1.0, L).astype(np.float32))


@functools.partial(
    pl.kernel,
    out_type=(
        jax.ShapeDtypeStruct((E,), jnp.float32),
        jax.ShapeDtypeStruct((E * L,), jnp.int32),
    ),
    mesh=_SC_MESH,
    compiler_params=pltpu.CompilerParams(needs_layout_passes=False),
    scratch_types=[
        pltpu.VMEM((N,), jnp.float32),
        pltpu.VMEM((CE,), jnp.int32),
        pltpu.VMEM((CE,), jnp.int32),
        pltpu.VMEM((CE,), jnp.float32),
        pltpu.VMEM((CE * L,), jnp.int32),
    ],
)
def _restrict_sc(u_hbm, v_hbm, b_hbm, red_hbm, inds_hbm,
                 btab, ubuf, vbuf, redbuf, indsbuf):
    wid = lax.axis_index("s") * 2 + lax.axis_index("c")
    base = wid * EPW
    pltpu.sync_copy(b_hbm, btab)

    def chunk_body(ci, _):
        cbase = base + ci * CE
        pltpu.sync_copy(u_hbm.at[pl.ds(cbase, CE)], ubuf)
        pltpu.sync_copy(v_hbm.at[pl.ds(cbase, CE)], vbuf)

        def grp(g, _):
            lane = lax.iota(jnp.int32, 16)
            lane60 = lane * L
            eloc = g * 16
            u = ubuf[pl.ds(eloc, 16)]
            vv = vbuf[pl.ds(eloc, 16)]
            fx = (u >> 7).astype(jnp.float32)
            fy = (u & (W - 1)).astype(jnp.float32)
            tx = (vv >> 7).astype(jnp.float32)
            ty = (vv & (W - 1)).astype(jnp.float32)
            dxf = tx - fx
            dyf = ty - fy
            red = jnp.zeros((16,), jnp.float32)
            sbase = eloc * L + lane60
            for l in range(L):
                tl = _TVALS[l]
                sx = (fx + dxf * tl).astype(jnp.int32)
                sy = (fy + dyf * tl).astype(jnp.int32)
                idx = (sx << 7) + sy
                red = red + plsc.load_gather(btab, [idx])
                plsc.store_scatter(indsbuf, [sbase + l], idx)
            redbuf[pl.ds(eloc, 16)] = red
            return 0

        lax.fori_loop(0, CE // 16, grp, 0)
        pltpu.sync_copy(redbuf, red_hbm.at[pl.ds(cbase, CE)])
        pltpu.sync_copy(indsbuf, inds_hbm.at[pl.ds(cbase * L, CE * L)])
        return 0

    lax.fori_loop(0, NCHUNK, chunk_body, 0)


EL = N * NUM_LONG
MPW = EL // NW
MCH = MPW // 128


@functools.partial(
    pl.kernel,
    out_type=(
        jax.ShapeDtypeStruct((N, PROP_DIM), jnp.float32),
        jax.ShapeDtypeStruct((N, PROP_DIM), jnp.float32),
    ),
    mesh=_SC_MESH,
    compiler_params=pltpu.CompilerParams(needs_layout_passes=False),
    scratch_types=[
        pltpu.VMEM_SHARED((N, PROP_DIM), jnp.float32),
        pltpu.VMEM((MCH, 128), jnp.int32),
        pltpu.VMEM((128, PROP_DIM), jnp.float32),
    ],
)
def _long_sc(msg_hbm, idx_hbm, zero_hbm, p0_hbm, p1_hbm, acc, idxbuf, msgbuf):
    cc = lax.axis_index("c")
    ss = lax.axis_index("s")
    wid = ss * 2 + cc
    base = wid * MPW
    pltpu.sync_copy(idx_hbm.at[wid], idxbuf)
    pltpu.sync_copy(zero_hbm.at[pl.ds(ss * (N // 16), N // 16)],
                    acc.at[pl.ds(ss * (N // 16), N // 16)])
    plsc.subcore_barrier()

    def chunk(r, _):
        pltpu.sync_copy(msg_hbm.at[pl.ds(base + r * 128, 128)], msgbuf)
        pltpu.sync_copy(msgbuf, acc.at[idxbuf.at[r]], add=True)
        return 0

    lax.fori_loop(0, MCH, chunk, 0)
    plsc.subcore_barrier()

    sl = pl.ds(ss * (N // 16), N // 16)

    @pl.when(cc == 0)
    def _():
        pltpu.sync_copy(acc.at[sl], p0_hbm.at[sl])

    @pl.when(cc == 1)
    def _():
        pltpu.sync_copy(acc.at[sl], p1_hbm.at[sl])


def _colshift_clip(y, dc):
    if dc == 0:
        return y
    pre = y.shape[:-1]
    if dc > 0:
        edge = jnp.sum(y[..., W - 1 - dc:], axis=-1, keepdims=True)
        return jnp.concatenate(
            [jnp.zeros(pre + (dc,), y.dtype), y[..., :W - 1 - dc], edge],
            axis=-1)
    d = -dc
    edge = jnp.sum(y[..., :d + 1], axis=-1, keepdims=True)
    return jnp.concatenate(
        [edge, y[..., d + 1:], jnp.zeros(pre + (d,), y.dtype)], axis=-1)


def _rowshift_clip(y, dr):
    if dr == 0:
        return y
    pre = y.shape[:-2]
    post = y.shape[-1:]
    if dr > 0:
        edge = jnp.sum(y[..., H - 1 - dr:, :], axis=-2, keepdims=True)
        return jnp.concatenate(
            [jnp.zeros(pre + (dr,) + post, y.dtype),
             y[..., :H - 1 - dr, :], edge], axis=-2)
    d = -dr
    edge = jnp.sum(y[..., :d + 1, :], axis=-2, keepdims=True)
    return jnp.concatenate(
        [edge, y[..., d + 1:, :], jnp.zeros(pre + (d,) + post, y.dtype)],
        axis=-2)


DB = 16


def _local_body(h_ref, wloc_ref, out_ref):
    h = h_ref[...]
    first = True
    k = 0
    for dr in range(-HALF, HALF + 1):
        acc = None
        for dc in range(-HALF, HALF + 1):
            y = wloc_ref[k][None] * h
            z = _colshift_clip(y, dc)
            acc = z if acc is None else acc + z
            k += 1
        r = _rowshift_clip(acc, dr)
        if first:
            out_ref[...] = r
            first = False
        else:
            out_ref[...] += r


def _softmax_body(x_ref, p_ref, hd_out_ref, hn_out_ref):
    x = x_ref[...] + p_ref[...]
    m = jnp.max(x, axis=0, keepdims=True)
    e = jnp.exp(x - m)
    s = jnp.sum(e, axis=0, keepdims=True)
    hd = e / s
    hd_out_ref[...] = hd
    hn_out_ref[...] = jnp.transpose(hd.reshape(PROP_DIM, N))


def _prop_step(hd, wloc, p_nodemajor):
    pd = jnp.transpose(p_nodemajor).reshape(PROP_DIM, H, W)
    pre = pl.pallas_call(
        _local_body,
        grid=(PROP_DIM // DB,),
        in_specs=[
            pl.BlockSpec((DB, H, W), lambda i: (i, 0, 0)),
            pl.BlockSpec((K * K, H, W), lambda i: (0, 0, 0)),
        ],
        out_specs=pl.BlockSpec((DB, H, W), lambda i: (i, 0, 0)),
        out_shape=jax.ShapeDtypeStruct((PROP_DIM, H, W), jnp.float32),
    )(hd, wloc)
    return pl.pallas_call(
        _softmax_body,
        out_shape=(
            jax.ShapeDtypeStruct((PROP_DIM, H, W), jnp.float32),
            jax.ShapeDtypeStruct((N, PROP_DIM), jnp.float32),
        ),
    )(pre, pd)


def _competition_body(flat_ref, agents_ref, aff_ref):
    flat = flat_ref[...]
    agents = agents_ref[...]
    scores = jax.lax.dot_general(
        flat, agents, (((1,), (1,)), ((), ())),
        preferred_element_type=jnp.float32)
    col = jax.lax.broadcasted_iota(jnp.int32, scores.shape, 1)
    valid = col < NUM_MASKS
    scores = jnp.where(valid, scores, -jnp.inf)
    m = jnp.max(scores, axis=-1, keepdims=True)
    e = jnp.where(valid, jnp.exp(scores - m), 0.0)
    masks = e / jnp.sum(e, axis=-1, keepdims=True)
    winner = jnp.max(masks, axis=0, keepdims=True)
    alive = jnp.where(winner > MASK_THRESH, 1.0, 0.0)
    aff_ref[...] = masks * alive


def _competition(flat, agents):
    agents_p = jnp.zeros((128, PROP_DIM), jnp.float32).at[:NUM_MASKS].set(agents)
    aff = pl.pallas_call(
        _competition_body,
        out_shape=jax.ShapeDtypeStruct((N, 128), jnp.float32),
    )(flat, agents_p)
    return aff[:, :NUM_MASKS]


def kernel(boundary):
    v, h0, agent_idx = _graph()

    u_flat = jnp.broadcast_to(
        jnp.arange(N, dtype=jnp.int32)[:, None], (N, KT)).reshape(-1)
    red, inds_flat = _restrict_sc(u_flat, v.reshape(-1), boundary.reshape(N))
    boundary_inds = inds_flat.reshape(1, N * KT, L)
    affinity = 10.0 - red.reshape(1, N, KT)

    adj = jax.nn.softmax(affinity, axis=-1)
    adj = adj / jnp.maximum(jnp.max(adj, axis=-1, keepdims=True), 1e-12)
    adj = adj[0]

    wloc = jnp.transpose(adj[:, :K * K]).reshape(K * K, H, W)
    wlong = adj[:, K * K:]
    vlong = v[:, K * K:].reshape(NW, MCH, 128)

    hd0 = jnp.transpose(h0[0]).reshape(PROP_DIM, H, W)
    zeros_nd = jnp.zeros((N, PROP_DIM), jnp.float32)

    def step(carry, _):
        hd, hn = carry
        msgs = (wlong[..., None] * hn[:, None, :]).reshape(-1, PROP_DIM)
        p0, p1 = _long_sc(msgs, vlong, zeros_nd)
        hd, hn = _prop_step(hd, wloc, p0 + p1)
        return (hd, hn), None

    (_, flat), _ = jax.lax.scan(step, (hd0, h0[0]), None, length=NUM_ITERS)

    prop_maps = flat[None]
    agents = jnp.take_along_axis(flat, agent_idx[0][:, None], axis=0)
    aff = _competition(flat, agents)
    aff_masks = aff.reshape(1, W, H, NUM_MASKS)
    return aff_masks, prop_maps, affinity, boundary_inds
